# Initial kernel scaffold; baseline (speedup 1.0000x reference)
import jax
import jax.numpy as jnp
from jax.experimental import pallas as pl


def kernel(x, W, b):
    raise NotImplementedError("write your pallas kernel here")


# speedup vs baseline: 1.0362x; 1.0362x over previous
import jax
import jax.numpy as jnp
from jax.experimental import pallas as pl

N_EMBD = 4096
N_HEAD = 32
LANES = 128
BM = 512


def _router_kernel(x_ref, wt_ref, b_ref, gate_ref, idx_ref):
    logits = jnp.dot(x_ref[...], wt_ref[...], preferred_element_type=jnp.float32)
    logits = logits + b_ref[...]
    lane = jax.lax.broadcasted_iota(jnp.int32, logits.shape, 1)
    neg = jnp.float32(-jnp.inf)
    l1 = jnp.where(lane < N_HEAD, logits, neg)
    m1 = jnp.max(l1, axis=1, keepdims=True)
    i1 = jnp.argmax(l1, axis=1).astype(jnp.int32)[:, None]
    l2 = jnp.where(lane == i1, neg, l1)
    m2 = jnp.max(l2, axis=1, keepdims=True)
    i2 = jnp.argmax(l2, axis=1).astype(jnp.int32)[:, None]
    p1 = 1.0 / (1.0 + jnp.exp(m2 - m1))
    p2 = 1.0 - p1
    zero = jnp.zeros_like(logits)
    gate_ref[...] = jnp.where(lane == i1, p1, jnp.where(lane == i2, p2, zero))
    idx_ref[...] = jnp.where(lane == 0, i1, jnp.where(lane == 1, i2, 0))


def kernel(x, W, b):
    B, S, D = x.shape
    M = B * S
    xf = x.reshape(M, D)
    wt = jnp.zeros((D, LANES), dtype=W.dtype).at[:, :N_HEAD].set(W.T)
    bp = jnp.zeros((1, LANES), dtype=b.dtype).at[0, :N_HEAD].set(b)

    grid = (M // BM,)
    gate_p, idx_p = pl.pallas_call(
        _router_kernel,
        grid=grid,
        in_specs=[
            pl.BlockSpec((BM, D), lambda i: (i, 0)),
            pl.BlockSpec((D, LANES), lambda i: (0, 0)),
            pl.BlockSpec((1, LANES), lambda i: (0, 0)),
        ],
        out_specs=[
            pl.BlockSpec((BM, LANES), lambda i: (i, 0)),
            pl.BlockSpec((BM, LANES), lambda i: (i, 0)),
        ],
        out_shape=[
            jax.ShapeDtypeStruct((M, LANES), jnp.float32),
            jax.ShapeDtypeStruct((M, LANES), jnp.int32),
        ],
    )(xf, wt, bp)

    gate = gate_p[:, :N_HEAD].reshape(B, S, N_HEAD)
    indices = idx_p[:, :2].reshape(B, S, 2)
    return (gate, indices)


# speedup vs baseline: 1.0987x; 1.0603x over previous
import jax
import jax.numpy as jnp
from jax.experimental import pallas as pl

N_EMBD = 4096
N_HEAD = 32
BM = 512


def _router_kernel(x_ref, wt_ref, b_ref, gate_ref, idx_ref):
    logits = jnp.dot(x_ref[...], wt_ref[...], preferred_element_type=jnp.float32)
    logits = logits + b_ref[...]
    lane = jax.lax.broadcasted_iota(jnp.int32, logits.shape, 1)
    neg = jnp.float32(-jnp.inf)
    m1 = jnp.max(logits, axis=1, keepdims=True)
    i1 = jnp.argmax(logits, axis=1).astype(jnp.int32)[:, None]
    l2 = jnp.where(lane == i1, neg, logits)
    m2 = jnp.max(l2, axis=1, keepdims=True)
    i2 = jnp.argmax(l2, axis=1).astype(jnp.int32)[:, None]
    p1 = 1.0 / (1.0 + jnp.exp(m2 - m1))
    p2 = 1.0 - p1
    zero = jnp.zeros_like(logits)
    gate_ref[...] = jnp.where(lane == i1, p1, jnp.where(lane == i2, p2, zero))
    idx_ref[...] = jnp.concatenate([i1, i2], axis=1)


def kernel(x, W, b):
    B, S, D = x.shape
    M = B * S
    xf = x.reshape(M, D)
    wt = W.T
    bp = b.reshape(1, N_HEAD)

    grid = (M // BM,)
    gate, idx = pl.pallas_call(
        _router_kernel,
        grid=grid,
        in_specs=[
            pl.BlockSpec((BM, D), lambda i: (i, 0)),
            pl.BlockSpec((D, N_HEAD), lambda i: (0, 0)),
            pl.BlockSpec((1, N_HEAD), lambda i: (0, 0)),
        ],
        out_specs=[
            pl.BlockSpec((BM, N_HEAD), lambda i: (i, 0)),
            pl.BlockSpec((BM, 2), lambda i: (i, 0)),
        ],
        out_shape=[
            jax.ShapeDtypeStruct((M, N_HEAD), jnp.float32),
            jax.ShapeDtypeStruct((M, 2), jnp.int32),
        ],
    )(xf, wt, bp)

    return (gate.reshape(B, S, N_HEAD), idx.reshape(B, S, 2))
